# baseline (speedup 1.0000x reference)
import functools

import jax
import jax.numpy as jnp
from jax import lax
from jax.experimental import pallas as pl
from jax.experimental.pallas import tpu as pltpu
from jax.experimental.pallas import tpu_sc as plsc

_F = 26
_V = 100000
_E = 16
_D = 128
_B = 4096
_L = 20

_FP = 32
_T = _B * _L
_R = _T * _FP
_NC = 2
_NS = 16
_NW = _NC * _NS
_RPW = _R // _NW
_K = 128
_INNER = 20
_OUTER = _RPW // (_K * _INNER)
_CH = _K * _INNER


def _sc_gather_body(table_hbm, idx_hbm, out_hbm, idx_v, rows_v, sem):
    wid = lax.axis_index("s") * _NC + lax.axis_index("c")

    def chunk(i, carry):
        pltpu.sync_copy(idx_hbm.at[wid, i], idx_v)
        copies = []
        for j in range(_INNER):
            copies.append(
                pltpu.async_copy(
                    table_hbm.at[idx_v.at[j]],
                    rows_v.at[pl.ds(j * _K, _K)],
                    sem,
                )
            )
        for c in copies:
            c.wait()
        pltpu.sync_copy(rows_v, out_hbm.at[pl.ds(wid * _RPW + i * _CH, _CH)])
        return carry

    lax.fori_loop(0, _OUTER, chunk, 0)


@functools.partial(
    pl.kernel,
    out_type=jax.ShapeDtypeStruct((_R, _E), jnp.float32),
    mesh=plsc.VectorSubcoreMesh(
        core_axis_name="c", subcore_axis_name="s",
        num_cores=_NC, num_subcores=_NS,
    ),
    scratch_types=[
        pltpu.VMEM((_INNER, _K), jnp.int32),
        pltpu.VMEM((_CH, _E), jnp.float32),
        pltpu.SemaphoreType.DMA,
    ],
    compiler_params=pltpu.CompilerParams(use_tc_tiling_on_sc=False),
)
def _sc_gather(table_hbm, idx_hbm, out_hbm, idx_v, rows_v, sem):
    _sc_gather_body(table_hbm, idx_hbm, out_hbm, idx_v, rows_v, sem)


_BM = 1024


def _proj_body(x_ref, w_ref, b_ref, g_ref, bt_ref, o_ref):
    x = x_ref[...]
    h = jnp.dot(x, w_ref[...], preferred_element_type=jnp.float32) + b_ref[...]
    h = h * jax.nn.sigmoid(h)
    m = jnp.mean(h, axis=-1, keepdims=True)
    c = h - m
    v = jnp.mean(c * c, axis=-1, keepdims=True)
    o_ref[...] = c * lax.rsqrt(v + 1e-5) * g_ref[...] + bt_ref[...]


def _tc_project(x2d, w1p, b1, gamma, beta):
    return pl.pallas_call(
        _proj_body,
        grid=(_T // _BM,),
        in_specs=[
            pl.BlockSpec((_BM, _FP * _E), lambda i: (i, 0)),
            pl.BlockSpec((_FP * _E, _D), lambda i: (0, 0)),
            pl.BlockSpec((1, _D), lambda i: (0, 0)),
            pl.BlockSpec((1, _D), lambda i: (0, 0)),
            pl.BlockSpec((1, _D), lambda i: (0, 0)),
        ],
        out_specs=pl.BlockSpec((_BM, _D), lambda i: (i, 0)),
        out_shape=jax.ShapeDtypeStruct((_T, _D), jnp.float32),
    )(x2d, w1p, b1.reshape(1, _D), gamma.reshape(1, _D), beta.reshape(1, _D))


def kernel(sequence, tables, W1, b1, gamma, beta):
    table_flat = tables.reshape(_F * (_V + 1), _E)

    offs = jnp.arange(_F, dtype=jnp.int32) * (_V + 1)
    idx = jnp.clip(sequence, 0, _V).astype(jnp.int32)
    idx = jnp.transpose(idx, (0, 2, 1)).reshape(_T, _F) + offs[None, :]
    tok = jnp.arange(_T, dtype=jnp.int32)[:, None]
    padf = jnp.arange(_FP - _F, dtype=jnp.int32)[None, :]
    pad_idx = ((tok * (_FP - _F) + padf) % _F) * (_V + 1)
    idx = jnp.concatenate([idx, pad_idx], axis=1)
    idx = idx.reshape(_NW, _OUTER, _INNER, _K)

    gathered = _sc_gather(table_flat, idx)
    x2d = gathered.reshape(_T, _FP * _E)

    w1p = jnp.concatenate(
        [W1, jnp.zeros((_FP * _E - _F * _E, _D), W1.dtype)], axis=0)

    out = _tc_project(x2d, w1p, b1, gamma, beta)
    return out.reshape(_B, _L, _D)


# speedup vs baseline: 3.7249x; 3.7249x over previous
import functools

import jax
import jax.numpy as jnp
from jax import lax
from jax.experimental import pallas as pl
from jax.experimental.pallas import tpu as pltpu
from jax.experimental.pallas import tpu_sc as plsc

_F = 26
_V = 100000
_E = 16
_D = 128
_B = 4096
_L = 20

_VP = 100096
_FP = 32
_T = _B * _L
_NC = 2
_NS = 16
_NW = _NC * _NS
_TPW = _T // _NW
_K = 128
_INNER = 20
_CH = _K * _INNER
_TPCH = _CH // 8
_OUTER = _TPW // _TPCH


_VC = 5888
_NVB = _VP // _VC
_TROWS = _F * _VP // 8


def _detile_body(t_ref, o_ref):
    o_ref[...] = jnp.transpose(t_ref[0], (1, 0))


def _tc_detile(tt):
    return pl.pallas_call(
        _detile_body,
        grid=(_F, _NVB),
        in_specs=[pl.BlockSpec((1, _E, _VC), lambda f, c: (f, 0, c))],
        out_specs=pl.BlockSpec((_VC, _E), lambda f, c: (f * _NVB + c, 0)),
        out_shape=jax.ShapeDtypeStruct((_F * _VP, _E), jnp.float32),
    )(tt)


def _sc_gather_body(table_hbm, idx_hbm, outs, idx_v, rows_v, sem):
    wid = lax.axis_index("s") * _NC + lax.axis_index("c")

    for c in range(4):
        out_c = outs[c]

        def chunk(i, carry, c=c, out_c=out_c):
            pltpu.sync_copy(idx_hbm.at[wid, c, i], idx_v)
            copies = []
            for j in range(_INNER):
                copies.append(
                    pltpu.async_copy(
                        table_hbm.at[idx_v.at[j]],
                        rows_v.at[pl.ds(j * _K, _K)],
                        sem,
                    )
                )
            for cp in copies:
                cp.wait()
            pltpu.sync_copy(
                rows_v,
                out_c.at[pl.ds((wid * _TPW + i * _TPCH) * 8, _CH)],
            )
            return carry

        lax.fori_loop(0, _OUTER, chunk, 0)


@functools.partial(
    pl.kernel,
    out_type=tuple(
        jax.ShapeDtypeStruct((_T * 8, _E), jnp.float32) for _ in range(4)
    ),
    mesh=plsc.VectorSubcoreMesh(
        core_axis_name="c", subcore_axis_name="s",
        num_cores=_NC, num_subcores=_NS,
    ),
    scratch_types=[
        pltpu.VMEM((_INNER, _K), jnp.int32),
        pltpu.VMEM((_CH, _E), jnp.float32),
        pltpu.SemaphoreType.DMA,
    ],
    compiler_params=pltpu.CompilerParams(use_tc_tiling_on_sc=False),
)
def _sc_gather(table_hbm, idx_hbm, o0, o1, o2, o3, idx_v, rows_v, sem):
    _sc_gather_body(table_hbm, idx_hbm, (o0, o1, o2, o3), idx_v, rows_v, sem)


_BM = 1024


def _proj_body(x0, x1, x2, x3, w0, w1, w2, w3, b_ref, g_ref, bt_ref, o_ref):
    h = jnp.dot(x0[...], w0[...], preferred_element_type=jnp.float32)
    h += jnp.dot(x1[...], w1[...], preferred_element_type=jnp.float32)
    h += jnp.dot(x2[...], w2[...], preferred_element_type=jnp.float32)
    h += jnp.dot(x3[...], w3[...], preferred_element_type=jnp.float32)
    h += b_ref[...]
    h = h * jax.nn.sigmoid(h)
    m = jnp.mean(h, axis=-1, keepdims=True)
    cen = h - m
    v = jnp.mean(cen * cen, axis=-1, keepdims=True)
    o_ref[...] = cen * lax.rsqrt(v + 1e-5) * g_ref[...] + bt_ref[...]


def _tc_project(xs, ws, b1, gamma, beta):
    xspec = pl.BlockSpec((_BM, _D), lambda i: (i, 0))
    wspec = pl.BlockSpec((_D, _D), lambda i: (0, 0))
    vspec = pl.BlockSpec((1, _D), lambda i: (0, 0))
    return pl.pallas_call(
        _proj_body,
        grid=(_T // _BM,),
        in_specs=[xspec] * 4 + [wspec] * 4 + [vspec] * 3,
        out_specs=pl.BlockSpec((_BM, _D), lambda i: (i, 0)),
        out_shape=jax.ShapeDtypeStruct((_T, _D), jnp.float32),
    )(*xs, *ws, b1.reshape(1, _D), gamma.reshape(1, _D), beta.reshape(1, _D))


def kernel(sequence, tables, W1, b1, gamma, beta):
    tt = jnp.transpose(tables, (0, 2, 1))
    table_rows = _tc_detile(tt)

    offs = jnp.arange(_F, dtype=jnp.int32) * _VP
    idx = jnp.clip(sequence, 0, _V).astype(jnp.int32)
    idx = jnp.transpose(idx, (0, 2, 1)).reshape(_T, _F) + offs[None, :]
    tok = jnp.arange(_T, dtype=jnp.int32)[:, None]
    padf = jnp.arange(_FP - _F, dtype=jnp.int32)[None, :]
    pad_idx = (tok * (_FP - _F) + padf) % 90000
    idx = jnp.concatenate([idx, pad_idx], axis=1)
    idx = idx.reshape(_NW, _OUTER, _TPCH, 4, 8)
    idx = jnp.transpose(idx, (0, 3, 1, 2, 4))
    idx = idx.reshape(_NW, 4, _OUTER, _INNER, _K)

    g0, g1, g2, g3 = _sc_gather(table_rows, idx)
    xs = [g.reshape(_T, _D) for g in (g0, g1, g2, g3)]

    w1p = jnp.concatenate(
        [W1, jnp.zeros((_FP * _E - _F * _E, _D), W1.dtype)], axis=0)
    ws = [w1p[c * _D:(c + 1) * _D] for c in range(4)]

    out = _tc_project(xs, ws, b1, gamma, beta)
    return out.reshape(_B, _L, _D)


# speedup vs baseline: 10.4329x; 2.8009x over previous
import functools

import jax
import jax.numpy as jnp
from jax import lax
from jax.experimental import pallas as pl
from jax.experimental.pallas import tpu as pltpu
from jax.experimental.pallas import tpu_sc as plsc

_F = 26
_V = 100000
_E = 16
_D = 128
_B = 4096
_L = 20

_VP = 100096
_FP = 32
_T = _B * _L
_NC = 2
_NS = 16
_NW = _NC * _NS
_TPW = _T // _NW
_K = 128
_INNER = 20
_CH = _K * _INNER
_TPCH = _CH // 8
_OUTER = _TPW // _TPCH


_VC = 5888
_NVB = _VP // _VC
_NFB = _FP // 8
_TROWS = _NFB * _VP


def _detile_body(t_ref, o_ref):
    x = t_ref[...]
    x = x.reshape(8 * _E, _VC)
    o_ref[...] = jnp.transpose(x, (1, 0))


def _tc_detile(tt):
    return pl.pallas_call(
        _detile_body,
        grid=(_NFB, _NVB),
        in_specs=[pl.BlockSpec((8, _E, _VC), lambda fb, c: (fb, 0, c))],
        out_specs=pl.BlockSpec((_VC, 128), lambda fb, c: (fb * _NVB + c, 0)),
        out_shape=jax.ShapeDtypeStruct((_TROWS, 128), jnp.float32),
    )(tt)


def _sc_gather_body(table_hbm, idx_hbm, outs, idx_v, rows_v, sem):
    wid = lax.axis_index("s") * _NC + lax.axis_index("c")

    for c in range(4):
        out_c = outs[c]

        def chunk(i, carry, c=c, out_c=out_c):
            pltpu.sync_copy(idx_hbm.at[wid, c, i], idx_v)
            copies = []
            for j in range(_INNER):
                copies.append(
                    pltpu.async_copy(
                        table_hbm.at[idx_v.at[j]],
                        rows_v.at[pl.ds(j * _K, _K)],
                        sem,
                    )
                )
            for cp in copies:
                cp.wait()
            pltpu.sync_copy(
                rows_v,
                out_c.at[pl.ds((wid * _TPW + i * _TPCH) * 8, _CH)],
            )
            return carry

        lax.fori_loop(0, _OUTER, chunk, 0)


@functools.partial(
    pl.kernel,
    out_type=tuple(
        jax.ShapeDtypeStruct((_T * 8, _E), jnp.float32) for _ in range(4)
    ),
    mesh=plsc.VectorSubcoreMesh(
        core_axis_name="c", subcore_axis_name="s",
        num_cores=_NC, num_subcores=_NS,
    ),
    scratch_types=[
        pltpu.VMEM((_INNER, _K), jnp.int32),
        pltpu.VMEM((_CH, _E), jnp.float32),
        pltpu.SemaphoreType.DMA,
    ],
    compiler_params=pltpu.CompilerParams(use_tc_tiling_on_sc=False),
)
def _sc_gather(table_hbm, idx_hbm, o0, o1, o2, o3, idx_v, rows_v, sem):
    _sc_gather_body(table_hbm, idx_hbm, (o0, o1, o2, o3), idx_v, rows_v, sem)


_BM = 1024


def _proj_body(x0, x1, x2, x3, w0, w1, w2, w3, b_ref, g_ref, bt_ref, o_ref):
    h = jnp.dot(x0[...], w0[...], preferred_element_type=jnp.float32)
    h += jnp.dot(x1[...], w1[...], preferred_element_type=jnp.float32)
    h += jnp.dot(x2[...], w2[...], preferred_element_type=jnp.float32)
    h += jnp.dot(x3[...], w3[...], preferred_element_type=jnp.float32)
    h += b_ref[...]
    h = h * jax.nn.sigmoid(h)
    m = jnp.mean(h, axis=-1, keepdims=True)
    cen = h - m
    v = jnp.mean(cen * cen, axis=-1, keepdims=True)
    o_ref[...] = cen * lax.rsqrt(v + 1e-5) * g_ref[...] + bt_ref[...]


def _tc_project(xs, ws, b1, gamma, beta):
    xspec = pl.BlockSpec((_BM, _D), lambda i: (i, 0))
    wspec = pl.BlockSpec((_D, _D), lambda i: (0, 0))
    vspec = pl.BlockSpec((1, _D), lambda i: (0, 0))
    return pl.pallas_call(
        _proj_body,
        grid=(_T // _BM,),
        in_specs=[xspec] * 4 + [wspec] * 4 + [vspec] * 3,
        out_specs=pl.BlockSpec((_BM, _D), lambda i: (i, 0)),
        out_shape=jax.ShapeDtypeStruct((_T, _D), jnp.float32),
    )(*xs, *ws, b1.reshape(1, _D), gamma.reshape(1, _D), beta.reshape(1, _D))


def kernel(sequence, tables, W1, b1, gamma, beta):
    tt = jnp.transpose(tables, (0, 2, 1))
    tlin = _tc_detile(tt)
    table_rows = tlin.reshape(_TROWS * 8, _E)

    farange = jnp.arange(_F, dtype=jnp.int32)
    offs = (farange // 8) * (_VP * 8) + (farange % 8)
    idx = jnp.clip(sequence, 0, _V).astype(jnp.int32)
    idx = jnp.transpose(idx, (0, 2, 1)).reshape(_T, _F) * 8 + offs[None, :]
    tok = jnp.arange(_T, dtype=jnp.int32)[:, None]
    padf = jnp.arange(_FP - _F, dtype=jnp.int32)[None, :]
    pad_idx = ((tok * (_FP - _F) + padf) % 90000) * 8
    idx = jnp.concatenate([idx, pad_idx], axis=1)
    idx = idx.reshape(_NW, _OUTER, _TPCH, 4, 8)
    idx = jnp.transpose(idx, (0, 3, 1, 2, 4))
    idx = idx.reshape(_NW, 4, _OUTER, _INNER, _K)

    g0, g1, g2, g3 = _sc_gather(table_rows, idx)
    xs = [g.reshape(_T, _D) for g in (g0, g1, g2, g3)]

    w1p = jnp.concatenate(
        [W1, jnp.zeros((_FP * _E - _F * _E, _D), W1.dtype)], axis=0)
    ws = [w1p[c * _D:(c + 1) * _D] for c in range(4)]

    out = _tc_project(xs, ws, b1, gamma, beta)
    return out.reshape(_B, _L, _D)


# speedup vs baseline: 13.6892x; 1.3121x over previous
import functools

import jax
import jax.numpy as jnp
from jax import lax
from jax.experimental import pallas as pl
from jax.experimental.pallas import tpu as pltpu
from jax.experimental.pallas import tpu_sc as plsc

_F = 26
_V = 100000
_E = 16
_D = 128
_B = 4096
_L = 20

_VP = 100096
_FP = 32
_T = _B * _L
_NC = 2
_NS = 16
_NW = _NC * _NS
_TPW = _T // _NW
_K = 128
_TKC = 256
_NHALF = _TKC // _K
_OUTER = _TPW // _TKC


_VC = 5888
_NVB = _VP // _VC
_NFB = _FP // 8
_TROWS = _NFB * _VP


def _detile_body(t_ref, o_ref):
    x = t_ref[...]
    x = x.reshape(8 * _E, _VC)
    o_ref[...] = jnp.transpose(x, (1, 0))


def _tc_detile(tt):
    return pl.pallas_call(
        _detile_body,
        grid=(_NFB, _NVB),
        in_specs=[pl.BlockSpec((8, _E, _VC), lambda fb, c: (fb, 0, c))],
        out_specs=pl.BlockSpec((_VC, 128), lambda fb, c: (fb * _NVB + c, 0)),
        out_shape=jax.ShapeDtypeStruct((_TROWS, 128), jnp.float32),
    )(tt)


def _sc_gather_body(table_hbm, idx_hbm, outs, idx_v, rows_v, sem):
    wid = lax.axis_index("s") * _NC + lax.axis_index("c")

    for c in range(4):
        out_c = outs[c]

        def chunk(i, carry, c=c, out_c=out_c):
            tok0 = wid * _TPW + i * _TKC
            l0 = tok0 // _B
            b0 = tok0 % _B
            pltpu.sync_copy(
                idx_hbm.at[pl.ds(c * 8, 8), l0, pl.ds(b0, _TKC)], idx_v)
            copies = []
            for fl in range(8):
                for k in range(_NHALF):
                    copies.append(
                        pltpu.async_copy(
                            table_hbm.at[idx_v.at[fl, pl.ds(k * _K, _K)]],
                            rows_v.at[fl, pl.ds(k * _K, _K)],
                            sem,
                        )
                    )
            for cp in copies:
                cp.wait()
            for fl in range(8):
                pltpu.sync_copy(
                    rows_v.at[fl],
                    out_c.at[pl.ds(tok0, _TKC), pl.ds(fl * _E, _E)])
            return carry

        lax.fori_loop(0, _OUTER, chunk, 0)


@functools.partial(
    pl.kernel,
    out_type=tuple(
        jax.ShapeDtypeStruct((_T, _D), jnp.float32) for _ in range(4)
    ),
    mesh=plsc.VectorSubcoreMesh(
        core_axis_name="c", subcore_axis_name="s",
        num_cores=_NC, num_subcores=_NS,
    ),
    scratch_types=[
        pltpu.VMEM((8, _TKC), jnp.int32),
        pltpu.VMEM((8, _TKC, _E), jnp.float32),
        pltpu.SemaphoreType.DMA,
    ],
    compiler_params=pltpu.CompilerParams(use_tc_tiling_on_sc=False),
)
def _sc_gather(table_hbm, idx_hbm, o0, o1, o2, o3, idx_v, rows_v, sem):
    _sc_gather_body(table_hbm, idx_hbm, (o0, o1, o2, o3), idx_v, rows_v, sem)


_BM = 1024


def _proj_body(x0, x1, x2, x3, w0, w1, w2, w3, b_ref, g_ref, bt_ref, o_ref):
    h = jnp.dot(x0[...], w0[...], preferred_element_type=jnp.float32)
    h += jnp.dot(x1[...], w1[...], preferred_element_type=jnp.float32)
    h += jnp.dot(x2[...], w2[...], preferred_element_type=jnp.float32)
    h += jnp.dot(x3[...], w3[...], preferred_element_type=jnp.float32)
    h += b_ref[...]
    h = h * jax.nn.sigmoid(h)
    m = jnp.mean(h, axis=-1, keepdims=True)
    cen = h - m
    v = jnp.mean(cen * cen, axis=-1, keepdims=True)
    o_ref[...] = cen * lax.rsqrt(v + 1e-5) * g_ref[...] + bt_ref[...]


def _tc_project(xs, ws, b1, gamma, beta):
    xspec = pl.BlockSpec((_BM, _D), lambda i: (i, 0))
    wspec = pl.BlockSpec((_D, _D), lambda i: (0, 0))
    vspec = pl.BlockSpec((1, _D), lambda i: (0, 0))
    return pl.pallas_call(
        _proj_body,
        grid=(_T // _BM,),
        in_specs=[xspec] * 4 + [wspec] * 4 + [vspec] * 3,
        out_specs=pl.BlockSpec((_BM, _D), lambda i: (i, 0)),
        out_shape=jax.ShapeDtypeStruct((_T, _D), jnp.float32),
    )(*xs, *ws, b1.reshape(1, _D), gamma.reshape(1, _D), beta.reshape(1, _D))


def kernel(sequence, tables, W1, b1, gamma, beta):
    tt = jnp.transpose(tables, (0, 2, 1))
    tlin = _tc_detile(tt)
    table_rows = tlin.reshape(_TROWS * 8, _E)

    seq_fm = jnp.transpose(sequence, (1, 2, 0))
    farange = jnp.arange(_F, dtype=jnp.int32)[:, None, None]
    offs = (farange // 8) * (_VP * 8) + (farange % 8)
    idx_real = jnp.clip(seq_fm, 0, _V).astype(jnp.int32) * 8 + offs
    padj = jnp.arange(_FP - _F, dtype=jnp.int32)[:, None, None]
    tok = (jnp.arange(_L, dtype=jnp.int32)[None, :, None] * _B
           + jnp.arange(_B, dtype=jnp.int32)[None, None, :])
    pad_idx = ((padj * _T + tok) % 90000) * 8
    idx = jnp.concatenate([idx_real, pad_idx], axis=0)

    xs = list(_sc_gather(table_rows, idx))

    w1p = jnp.concatenate(
        [W1, jnp.zeros((_FP * _E - _F * _E, _D), W1.dtype)], axis=0)
    ws = [w1p[c * _D:(c + 1) * _D] for c in range(4)]

    out = _tc_project(xs, ws, b1, gamma, beta)
    return jnp.transpose(out.reshape(_L, _B, _D), (1, 0, 2))
